# Initial kernel scaffold; baseline (speedup 1.0000x reference)
import jax
import jax.numpy as jnp
from jax.experimental import pallas as pl


def kernel(inputs, table):
    raise NotImplementedError("write your pallas kernel here")


# speedup vs baseline: 12.5181x; 12.5181x over previous
import functools

import jax
import jax.numpy as jnp
from jax import lax
from jax.experimental import pallas as pl
from jax.experimental.pallas import tpu as pltpu
from jax.experimental.pallas import tpu_sc as plsc

BATCH = 4096
SLOT = 26
NNZ = 20
VOCAB = 1000000
DIM = 32

NC = 2
NS = 16
NW = NC * NS
LANES = 16

SEGS = BATCH * SLOT
SPW = SEGS // NW
CSEG = 64
CKEY = CSEG * NNZ
NCHUNK = SPW // CSEG
GCHUNK = 128
NGATHER = CKEY // GCHUNK


def _emb_kernel(table_hbm, keys_hbm, out_hbm, idx_v, rows_v, out_v, sem):
    wid = lax.axis_index("s") * NC + lax.axis_index("c")
    seg_base = wid * SPW
    key_base = seg_base * NNZ

    @pl.loop(0, NCHUNK)
    def chunk_body(c):
        kbase = key_base + c * CKEY
        pltpu.sync_copy(keys_hbm.at[pl.ds(kbase, CKEY)], idx_v)
        copies = []
        for j in range(NGATHER):
            copies.append(pltpu.async_copy(
                table_hbm.at[idx_v.at[pl.ds(j * GCHUNK, GCHUNK)]],
                rows_v.at[pl.ds(j * GCHUNK, GCHUNK)],
                sem))
        for cp in copies:
            cp.wait()

        @pl.loop(0, CSEG)
        def seg_body(s):
            r = s * NNZ
            acc0 = rows_v[r, pl.ds(0, LANES)]
            acc1 = rows_v[r, pl.ds(LANES, LANES)]
            for j in range(1, NNZ):
                acc0 = acc0 + rows_v[r + j, pl.ds(0, LANES)]
                acc1 = acc1 + rows_v[r + j, pl.ds(LANES, LANES)]
            out_v[s, pl.ds(0, LANES)] = acc0
            out_v[s, pl.ds(LANES, LANES)] = acc1

        pltpu.sync_copy(out_v, out_hbm.at[pl.ds(seg_base + c * CSEG, CSEG)])


@jax.jit
def _run(keys, table):
    mesh = plsc.VectorSubcoreMesh(
        core_axis_name="c", subcore_axis_name="s",
        num_cores=NC, num_subcores=NS)
    f = pl.kernel(
        _emb_kernel,
        out_type=jax.ShapeDtypeStruct((SEGS, DIM), jnp.float32),
        mesh=mesh,
        scratch_types=[
            pltpu.VMEM((CKEY,), jnp.int32),
            pltpu.VMEM((CKEY, DIM), jnp.float32),
            pltpu.VMEM((CSEG, DIM), jnp.float32),
            pltpu.SemaphoreType.DMA,
        ],
        compiler_params=pltpu.CompilerParams(use_tc_tiling_on_sc=False),
    )
    return f(table, keys)


def kernel(inputs, table):
    keys = inputs.reshape(-1)
    out = _run(keys, table)
    return out.reshape(BATCH, SLOT, DIM)


# speedup vs baseline: 14.3607x; 1.1472x over previous
import functools

import jax
import jax.numpy as jnp
from jax import lax
from jax.experimental import pallas as pl
from jax.experimental.pallas import tpu as pltpu
from jax.experimental.pallas import tpu_sc as plsc

BATCH = 4096
SLOT = 26
NNZ = 20
VOCAB = 1000000
DIM = 32

NC = 2
NS = 16
NW = NC * NS
LANES = 16

SEGS = BATCH * SLOT
SPW = SEGS // NW
CSEG = 64
CKEY = CSEG * NNZ
NCHUNK = SPW // CSEG
GCHUNK = 128
NGATHER = CKEY // GCHUNK


def _emb_kernel(table_hbm, keys_hbm, out_hbm,
                idx0, idx1, rows0, rows1, out0, out1, sem0, sem1):
    wid = lax.axis_index("s") * NC + lax.axis_index("c")
    seg_base = wid * SPW
    key_base = seg_base * NNZ
    idx_v = (idx0, idx1)
    rows_v = (rows0, rows1)
    out_v = (out0, out1)
    sems = (sem0, sem1)

    def fire(chunk, b):
        pltpu.sync_copy(keys_hbm.at[pl.ds(key_base + chunk * CKEY, CKEY)],
                        idx_v[b])
        for j in range(NGATHER):
            pltpu.async_copy(
                table_hbm.at[idx_v[b].at[pl.ds(j * GCHUNK, GCHUNK)]],
                rows_v[b].at[pl.ds(j * GCHUNK, GCHUNK)],
                sems[b])

    def drain(b):
        for j in range(NGATHER):
            pltpu.make_async_copy(
                table_hbm.at[idx_v[b].at[pl.ds(j * GCHUNK, GCHUNK)]],
                rows_v[b].at[pl.ds(j * GCHUNK, GCHUNK)],
                sems[b]).wait()

    def reduce_store(chunk, b):
        rows = rows_v[b]
        out = out_v[b]

        @pl.loop(0, CSEG)
        def seg_body(s):
            r = s * NNZ
            acc0 = rows[r, pl.ds(0, LANES)]
            acc1 = rows[r, pl.ds(LANES, LANES)]
            for j in range(1, NNZ):
                acc0 = acc0 + rows[r + j, pl.ds(0, LANES)]
                acc1 = acc1 + rows[r + j, pl.ds(LANES, LANES)]
            out[s, pl.ds(0, LANES)] = acc0
            out[s, pl.ds(LANES, LANES)] = acc1

        pltpu.sync_copy(out, out_hbm.at[pl.ds(seg_base + chunk * CSEG, CSEG)])

    fire(0, 0)

    @pl.loop(0, NCHUNK, step=2)
    def chunk_body(cc):
        for b in range(2):
            c = cc + b
            nxt = c + 1

            @pl.when(nxt < NCHUNK)
            def _():
                fire(nxt, 1 - b)

            drain(b)
            reduce_store(c, b)


@jax.jit
def _run(keys, table):
    mesh = plsc.VectorSubcoreMesh(
        core_axis_name="c", subcore_axis_name="s",
        num_cores=NC, num_subcores=NS)
    f = pl.kernel(
        _emb_kernel,
        out_type=jax.ShapeDtypeStruct((SEGS, DIM), jnp.float32),
        mesh=mesh,
        scratch_types=[
            pltpu.VMEM((CKEY,), jnp.int32),
            pltpu.VMEM((CKEY,), jnp.int32),
            pltpu.VMEM((CKEY, DIM), jnp.float32),
            pltpu.VMEM((CKEY, DIM), jnp.float32),
            pltpu.VMEM((CSEG, DIM), jnp.float32),
            pltpu.VMEM((CSEG, DIM), jnp.float32),
            pltpu.SemaphoreType.DMA,
            pltpu.SemaphoreType.DMA,
        ],
        compiler_params=pltpu.CompilerParams(use_tc_tiling_on_sc=False),
    )
    return f(table, keys)


def kernel(inputs, table):
    keys = inputs.reshape(-1)
    out = _run(keys, table)
    return out.reshape(BATCH, SLOT, DIM)
